# baseline (speedup 1.0000x reference)
import functools

import jax
import jax.numpy as jnp
from jax import lax
from jax.experimental import pallas as pl
from jax.experimental.pallas import tpu as pltpu
from jax.experimental.pallas import tpu_sc as plsc

N = 10000
D = 128
E = 320000

NC = 2
NS = 16
NW = NC * NS

N_PAD = 10240
ROWS_PER_TILE = N_PAD // NS
E_PAD = 327680
CHUNKS = 80
CE = 128

BM = 512
GRID = N_PAD // BM

_mesh = plsc.VectorSubcoreMesh(core_axis_name="c", subcore_axis_name="s")


@functools.partial(
    pl.kernel,
    out_type=jax.ShapeDtypeStruct((NC, N_PAD), jnp.float32),
    mesh=_mesh,
    scratch_types=[
        pltpu.VMEM((CHUNKS, CE), jnp.int32),
        pltpu.VMEM((CHUNKS, CE), jnp.float32),
        pltpu.VMEM((ROWS_PER_TILE,), jnp.float32),
        pltpu.VMEM_SHARED((N_PAD,), jnp.float32),
    ],
)
def _deg_kernel(dst_hbm, ew_hbm, out_hbm, dstb, ewb, zv, degsh):
    c = lax.axis_index("c")
    s = lax.axis_index("s")
    wid = c * NS + s
    z16 = jnp.zeros((16,), jnp.float32)
    for k in range(ROWS_PER_TILE // 16):
        zv[pl.ds(k * 16, 16)] = z16
    pltpu.sync_copy(zv, degsh.at[pl.ds(s * ROWS_PER_TILE, ROWS_PER_TILE)])
    plsc.subcore_barrier()
    pltpu.sync_copy(dst_hbm.at[wid], dstb)
    pltpu.sync_copy(ew_hbm.at[wid], ewb)

    def body(j, _):
        pltpu.sync_copy(ewb.at[j], degsh.at[dstb.at[j]], add=True)
        return 0

    lax.fori_loop(0, CHUNKS, body, 0)
    plsc.subcore_barrier()
    pltpu.sync_copy(degsh.at[pl.ds(s * ROWS_PER_TILE, ROWS_PER_TILE)],
                    out_hbm.at[c, pl.ds(s * ROWS_PER_TILE, ROWS_PER_TILE)])


@functools.partial(
    pl.kernel,
    out_type=jax.ShapeDtypeStruct((NC, N_PAD, D), jnp.float32),
    mesh=_mesh,
    scratch_types=[
        pltpu.VMEM((2, CE), jnp.int32),
        pltpu.VMEM((2, CE), jnp.int32),
        pltpu.VMEM((2, CE), jnp.int32),
        pltpu.VMEM((2, CE), jnp.int32),
        pltpu.VMEM((CE,), jnp.float32),
        pltpu.VMEM((CE,), jnp.float32),
        pltpu.VMEM((CE,), jnp.float32),
        pltpu.VMEM((CE,), jnp.float32),
        pltpu.VMEM((CE, D), jnp.float32),
        pltpu.VMEM((CE, D), jnp.float32),
        pltpu.VMEM_SHARED((N_PAD, D), jnp.float32),
        pltpu.SemaphoreType.DMA,
        pltpu.SemaphoreType.DMA,
        pltpu.SemaphoreType.DMA,
        pltpu.SemaphoreType.DMA,
        pltpu.SemaphoreType.DMA,
        pltpu.SemaphoreType.DMA,
        pltpu.SemaphoreType.DMA,
        pltpu.SemaphoreType.DMA,
        pltpu.SemaphoreType.DMA,
        pltpu.SemaphoreType.DMA,
    ],
)
def _agg_kernel(pk_hbm, ew_hbm, g_hbm, out_hbm,
                ib0, ib1, ib2, ib3, eb0, eb1, eb2, eb3, r0, r1, accsh,
                isem0, isem1, isem2, isem3, esem0, esem1, esem2, esem3,
                gsem0, gsem1):
    c = lax.axis_index("c")
    s = lax.axis_index("s")
    wid = c * NS + s
    z16 = jnp.zeros((16,), jnp.float32)
    ibs = (ib0, ib1, ib2, ib3)
    isems = (isem0, isem1, isem2, isem3)
    ebs = (eb0, eb1, eb2, eb3)
    esems = (esem0, esem1, esem2, esem3)
    rbs = (r0, r1)
    gsems = (gsem0, gsem1)

    def zrow(r, _):
        for cg in range(D // 16):
            r0[r, pl.ds(cg * 16, 16)] = z16
        return 0

    lax.fori_loop(0, CE, zrow, 0)
    for k in range(ROWS_PER_TILE // CE):
        pltpu.sync_copy(
            r0, accsh.at[pl.ds(s * ROWS_PER_TILE + k * CE, CE)])
    plsc.subcore_barrier()

    def scale(eb, rb):
        def grp(g2, _):
            for half in range(2):
                gi = g2 * 2 + half
                ew16 = eb[pl.ds(gi * 16, 16)]
                for lane in range(16):
                    e = gi * 16 + lane
                    w = ew16.at[jnp.full((16,), lane, jnp.int32)].get(
                        mode="promise_in_bounds")
                    for cg in range(D // 16):
                        sl = pl.ds(cg * 16, 16)
                        rb[e, sl] = rb[e, sl] * w
            return 0

        lax.fori_loop(0, CE // 32, grp, 0)

    for b in range(4):
        pltpu.async_copy(pk_hbm.at[wid, b], ibs[b], isems[b])
        pltpu.async_copy(ew_hbm.at[wid, b], ebs[b], esems[b])
    pltpu.make_async_copy(pk_hbm.at[wid, 0], ib0, isem0).wait()
    pltpu.async_copy(g_hbm.at[ib0.at[0]], r0, gsem0)
    pltpu.make_async_copy(pk_hbm.at[wid, 0], ib1, isem1).wait()
    pltpu.async_copy(g_hbm.at[ib1.at[0]], r1, gsem1)

    def stage(j, i):
        ib = ibs[i]
        ib2_ = ibs[(i + 2) % 4]
        rb = rbs[i % 2]
        pltpu.make_async_copy(pk_hbm.at[wid, 0], ib2_,
                              isems[(i + 2) % 4]).wait()
        pltpu.make_async_copy(ew_hbm.at[wid, 0], ebs[i], esems[i]).wait()
        pltpu.make_async_copy(g_hbm.at[ib.at[0]], rb, gsems[i % 2]).wait()
        scale(ebs[i], rb)
        pltpu.sync_copy(rb, accsh.at[ib.at[1]], add=True)
        pltpu.async_copy(g_hbm.at[ib2_.at[0]], rb, gsems[i % 2])
        nxt = jnp.minimum(j + 4, CHUNKS - 1)
        pltpu.async_copy(pk_hbm.at[wid, nxt], ib, isems[i])
        pltpu.async_copy(ew_hbm.at[wid, nxt], ebs[i], esems[i])

    def body(j4, _):
        for i in range(4):
            stage(j4 * 4 + i, i)
        return 0

    lax.fori_loop(0, CHUNKS // 4, body, 0)
    for i in ((CHUNKS - 2) % 4, (CHUNKS - 1) % 4):
        pltpu.make_async_copy(pk_hbm.at[wid, 0], ibs[i], isems[i]).wait()
    for i in (1, 2, 3, 0):
        pltpu.make_async_copy(ew_hbm.at[wid, 0], ebs[i], esems[i]).wait()
    pltpu.make_async_copy(g_hbm.at[ib0.at[0]], r0, gsem0).wait()
    pltpu.make_async_copy(g_hbm.at[ib1.at[0]], r1, gsem1).wait()
    plsc.subcore_barrier()
    pltpu.sync_copy(accsh.at[pl.ds(s * ROWS_PER_TILE, ROWS_PER_TILE)],
                    out_hbm.at[c, pl.ds(s * ROWS_PER_TILE, ROWS_PER_TILE)])


def _tc1_body(deg_ref, x_ref, w_ref, g_ref, dinv_ref):
    deg = deg_ref[0] + deg_ref[1] + 1.0
    dinv = lax.rsqrt(deg)
    h = jnp.dot(x_ref[...], w_ref[...],
                preferred_element_type=jnp.float32,
                precision=lax.Precision.HIGHEST)
    g_ref[...] = dinv * h
    dinv_ref[...] = dinv


def _tc1(deg2, x, w1):
    return pl.pallas_call(
        _tc1_body,
        grid=(GRID,),
        in_specs=[
            pl.BlockSpec((NC, BM, 1), lambda i: (0, i, 0)),
            pl.BlockSpec((BM, D), lambda i: (i, 0)),
            pl.BlockSpec((D, D), lambda i: (0, 0)),
        ],
        out_specs=[
            pl.BlockSpec((BM, D), lambda i: (i, 0)),
            pl.BlockSpec((BM, 1), lambda i: (i, 0)),
        ],
        out_shape=[
            jax.ShapeDtypeStruct((N_PAD, D), jnp.float32),
            jax.ShapeDtypeStruct((N_PAD, 1), jnp.float32),
        ],
    )(deg2, x, w1)


def _tc2_body(acc_ref, g_ref, dinv_ref, b_ref, a_ref, w_ref, out_ref):
    dinv = dinv_ref[...]
    pre = dinv * (acc_ref[0] + acc_ref[1] + g_ref[...]) + b_ref[...]
    h = jnp.where(pre >= 0.0, pre, a_ref[...] * pre)
    out_ref[...] = dinv * jnp.dot(h, w_ref[...],
                                  preferred_element_type=jnp.float32,
                                  precision=lax.Precision.HIGHEST)


def _tc2(acc, g, dinv, b, a, w2):
    return pl.pallas_call(
        _tc2_body,
        grid=(GRID,),
        in_specs=[
            pl.BlockSpec((NC, BM, D), lambda i: (0, i, 0)),
            pl.BlockSpec((BM, D), lambda i: (i, 0)),
            pl.BlockSpec((BM, 1), lambda i: (i, 0)),
            pl.BlockSpec((1, D), lambda i: (0, 0)),
            pl.BlockSpec((1, D), lambda i: (0, 0)),
            pl.BlockSpec((D, D), lambda i: (0, 0)),
        ],
        out_specs=pl.BlockSpec((BM, D), lambda i: (i, 0)),
        out_shape=jax.ShapeDtypeStruct((N_PAD, D), jnp.float32),
    )(acc, g, dinv, b, a, w2)


def _tc3_body(acc_ref, g_ref, dinv_ref, b_ref, a_ref, out_ref):
    pre = (dinv_ref[...] * (acc_ref[0] + acc_ref[1] + g_ref[...])
           + b_ref[...])
    out_ref[...] = jnp.where(pre >= 0.0, pre, a_ref[...] * pre)


def _tc3(acc, g, dinv, b, a):
    return pl.pallas_call(
        _tc3_body,
        grid=(GRID,),
        in_specs=[
            pl.BlockSpec((NC, BM, D), lambda i: (0, i, 0)),
            pl.BlockSpec((BM, D), lambda i: (i, 0)),
            pl.BlockSpec((BM, 1), lambda i: (i, 0)),
            pl.BlockSpec((1, D), lambda i: (0, 0)),
            pl.BlockSpec((1, D), lambda i: (0, 0)),
        ],
        out_specs=pl.BlockSpec((BM, D), lambda i: (i, 0)),
        out_shape=jax.ShapeDtypeStruct((N_PAD, D), jnp.float32),
    )(acc, g, dinv, b, a)


def kernel(features, edge_index, edge_weight, W1, b1, a1, W2, b2, a2):
    src = edge_index[0]
    dst = edge_index[1]
    pad = E_PAD - E
    src_r = jnp.concatenate(
        [src, jnp.zeros((pad,), jnp.int32)]).reshape(NW, CHUNKS, CE)
    dst_r = jnp.concatenate(
        [dst, jnp.zeros((pad,), jnp.int32)]).reshape(NW, CHUNKS, CE)
    ew_r = jnp.concatenate(
        [edge_weight, jnp.zeros((pad,), jnp.float32)]).reshape(NW, CHUNKS, CE)
    pk = jnp.stack([src_r, dst_r], axis=2)
    x = jnp.concatenate(
        [features, jnp.zeros((N_PAD - N, D), jnp.float32)], axis=0)

    deg2 = _deg_kernel(dst_r, ew_r).reshape(NC, N_PAD, 1)

    g1, dinv = _tc1(deg2, x, W1)
    acc1 = _agg_kernel(pk, ew_r, g1)
    g2 = _tc2(acc1, g1, dinv, b1.reshape(1, D), a1.reshape(1, D), W2)
    acc2 = _agg_kernel(pk, ew_r, g2)
    out = _tc3(acc2, g2, dinv, b2.reshape(1, D), a2.reshape(1, D))
    return out[:N]


# speedup vs baseline: 1.0203x; 1.0203x over previous
import functools

import jax
import jax.numpy as jnp
from jax import lax
from jax.experimental import pallas as pl
from jax.experimental.pallas import tpu as pltpu
from jax.experimental.pallas import tpu_sc as plsc

N = 10000
D = 128
E = 320000

NC = 2
NS = 16
NW = NC * NS

N_PAD = 10240
ROWS_PER_TILE = N_PAD // NS
E_PAD = 327680
CHUNKS = 80
CE = 128
TOTC = E_PAD // CE
CNT0 = 36
CNT1 = (TOTC - NS * CNT0) // NS

BM = 512
GRID = N_PAD // BM

_mesh = plsc.VectorSubcoreMesh(core_axis_name="c", subcore_axis_name="s")


@functools.partial(
    pl.kernel,
    out_type=jax.ShapeDtypeStruct((NC, N_PAD), jnp.float32),
    mesh=_mesh,
    scratch_types=[
        pltpu.VMEM((CHUNKS, CE), jnp.int32),
        pltpu.VMEM((CHUNKS, CE), jnp.float32),
        pltpu.VMEM((ROWS_PER_TILE,), jnp.float32),
        pltpu.VMEM_SHARED((N_PAD,), jnp.float32),
    ],
)
def _deg_kernel(dst_hbm, ew_hbm, out_hbm, dstb, ewb, zv, degsh):
    c = lax.axis_index("c")
    s = lax.axis_index("s")
    wid = c * NS + s
    z16 = jnp.zeros((16,), jnp.float32)
    for k in range(ROWS_PER_TILE // 16):
        zv[pl.ds(k * 16, 16)] = z16
    pltpu.sync_copy(zv, degsh.at[pl.ds(s * ROWS_PER_TILE, ROWS_PER_TILE)])
    plsc.subcore_barrier()
    pltpu.sync_copy(dst_hbm.at[wid], dstb)
    pltpu.sync_copy(ew_hbm.at[wid], ewb)

    def body(j, _):
        pltpu.sync_copy(ewb.at[j], degsh.at[dstb.at[j]], add=True)
        return 0

    lax.fori_loop(0, CHUNKS, body, 0)
    plsc.subcore_barrier()
    pltpu.sync_copy(degsh.at[pl.ds(s * ROWS_PER_TILE, ROWS_PER_TILE)],
                    out_hbm.at[c, pl.ds(s * ROWS_PER_TILE, ROWS_PER_TILE)])


@functools.partial(
    pl.kernel,
    out_type=jax.ShapeDtypeStruct((NC, N_PAD, D), jnp.float32),
    mesh=_mesh,
    scratch_types=[
        pltpu.VMEM((2, CE), jnp.int32),
        pltpu.VMEM((2, CE), jnp.int32),
        pltpu.VMEM((2, CE), jnp.int32),
        pltpu.VMEM((2, CE), jnp.int32),
        pltpu.VMEM((CE,), jnp.float32),
        pltpu.VMEM((CE,), jnp.float32),
        pltpu.VMEM((CE,), jnp.float32),
        pltpu.VMEM((CE,), jnp.float32),
        pltpu.VMEM((CE, D), jnp.float32),
        pltpu.VMEM((CE, D), jnp.float32),
        pltpu.VMEM_SHARED((N_PAD, D), jnp.float32),
        pltpu.SemaphoreType.DMA,
        pltpu.SemaphoreType.DMA,
        pltpu.SemaphoreType.DMA,
        pltpu.SemaphoreType.DMA,
        pltpu.SemaphoreType.DMA,
        pltpu.SemaphoreType.DMA,
        pltpu.SemaphoreType.DMA,
        pltpu.SemaphoreType.DMA,
        pltpu.SemaphoreType.DMA,
        pltpu.SemaphoreType.DMA,
    ],
)
def _agg_kernel(pk_hbm, ew_hbm, g_hbm, out_hbm,
                ib0, ib1, ib2, ib3, eb0, eb1, eb2, eb3, r0, r1, accsh,
                isem0, isem1, isem2, isem3, esem0, esem1, esem2, esem3,
                gsem0, gsem1):
    c = lax.axis_index("c")
    s = lax.axis_index("s")
    wid = c * NS + s
    z16 = jnp.zeros((16,), jnp.float32)
    ibs = (ib0, ib1, ib2, ib3)
    isems = (isem0, isem1, isem2, isem3)
    ebs = (eb0, eb1, eb2, eb3)
    esems = (esem0, esem1, esem2, esem3)
    rbs = (r0, r1)
    gsems = (gsem0, gsem1)

    def zrow(r, _):
        for cg in range(D // 16):
            r0[r, pl.ds(cg * 16, 16)] = z16
        return 0

    lax.fori_loop(0, CE, zrow, 0)
    for k in range(ROWS_PER_TILE // CE):
        pltpu.sync_copy(
            r0, accsh.at[pl.ds(s * ROWS_PER_TILE + k * CE, CE)])
    plsc.subcore_barrier()

    def scale(eb, rb):
        def grp(g2, _):
            for half in range(2):
                gi = g2 * 2 + half
                ew16 = eb[pl.ds(gi * 16, 16)]
                for lane in range(16):
                    e = gi * 16 + lane
                    w = ew16.at[jnp.full((16,), lane, jnp.int32)].get(
                        mode="promise_in_bounds")
                    for cg in range(D // 16):
                        sl = pl.ds(cg * 16, 16)
                        rb[e, sl] = rb[e, sl] * w
            return 0

        lax.fori_loop(0, CE // 32, grp, 0)

    def run(base, cnt):
        for b in range(4):
            pltpu.async_copy(pk_hbm.at[base + b], ibs[b], isems[b])
            pltpu.async_copy(ew_hbm.at[base + b], ebs[b], esems[b])
        pltpu.make_async_copy(pk_hbm.at[base], ib0, isem0).wait()
        pltpu.async_copy(g_hbm.at[ib0.at[0]], r0, gsem0)
        pltpu.make_async_copy(pk_hbm.at[base], ib1, isem1).wait()
        pltpu.async_copy(g_hbm.at[ib1.at[0]], r1, gsem1)

        def stage(j, i):
            ib = ibs[i]
            ib2_ = ibs[(i + 2) % 4]
            rb = rbs[i % 2]
            pltpu.make_async_copy(pk_hbm.at[base], ib2_,
                                  isems[(i + 2) % 4]).wait()
            pltpu.make_async_copy(ew_hbm.at[base], ebs[i], esems[i]).wait()
            pltpu.make_async_copy(g_hbm.at[ib.at[0]], rb,
                                  gsems[i % 2]).wait()
            scale(ebs[i], rb)
            pltpu.sync_copy(rb, accsh.at[ib.at[1]], add=True)
            pltpu.async_copy(g_hbm.at[ib2_.at[0]], rb, gsems[i % 2])
            nxt = base + jnp.minimum(j + 4, cnt - 1)
            pltpu.async_copy(pk_hbm.at[nxt], ib, isems[i])
            pltpu.async_copy(ew_hbm.at[nxt], ebs[i], esems[i])

        def body(j4, _):
            for i in range(4):
                stage(j4 * 4 + i, i)
            return 0

        lax.fori_loop(0, cnt // 4, body, 0)
        for i in ((cnt - 2) % 4, (cnt - 1) % 4):
            pltpu.make_async_copy(pk_hbm.at[base], ibs[i], isems[i]).wait()
        for i in (1, 2, 3, 0):
            pltpu.make_async_copy(ew_hbm.at[base], ebs[i], esems[i]).wait()
        pltpu.make_async_copy(g_hbm.at[ib0.at[0]], r0, gsem0).wait()
        pltpu.make_async_copy(g_hbm.at[ib1.at[0]], r1, gsem1).wait()

    @pl.when(c == 0)
    def _():
        run(s * CNT0, CNT0)

    @pl.when(c == 1)
    def _():
        run(NS * CNT0 + s * CNT1, CNT1)

    plsc.subcore_barrier()
    pltpu.sync_copy(accsh.at[pl.ds(s * ROWS_PER_TILE, ROWS_PER_TILE)],
                    out_hbm.at[c, pl.ds(s * ROWS_PER_TILE, ROWS_PER_TILE)])


def _tc1_body(deg_ref, x_ref, w_ref, g_ref, dinv_ref):
    deg = deg_ref[0] + deg_ref[1] + 1.0
    dinv = lax.rsqrt(deg)
    h = jnp.dot(x_ref[...], w_ref[...],
                preferred_element_type=jnp.float32,
                precision=lax.Precision.HIGHEST)
    g_ref[...] = dinv * h
    dinv_ref[...] = dinv


def _tc1(deg2, x, w1):
    return pl.pallas_call(
        _tc1_body,
        grid=(GRID,),
        in_specs=[
            pl.BlockSpec((NC, BM, 1), lambda i: (0, i, 0)),
            pl.BlockSpec((BM, D), lambda i: (i, 0)),
            pl.BlockSpec((D, D), lambda i: (0, 0)),
        ],
        out_specs=[
            pl.BlockSpec((BM, D), lambda i: (i, 0)),
            pl.BlockSpec((BM, 1), lambda i: (i, 0)),
        ],
        out_shape=[
            jax.ShapeDtypeStruct((N_PAD, D), jnp.float32),
            jax.ShapeDtypeStruct((N_PAD, 1), jnp.float32),
        ],
    )(deg2, x, w1)


def _tc2_body(acc_ref, g_ref, dinv_ref, b_ref, a_ref, w_ref, out_ref):
    dinv = dinv_ref[...]
    pre = dinv * (acc_ref[0] + acc_ref[1] + g_ref[...]) + b_ref[...]
    h = jnp.where(pre >= 0.0, pre, a_ref[...] * pre)
    out_ref[...] = dinv * jnp.dot(h, w_ref[...],
                                  preferred_element_type=jnp.float32,
                                  precision=lax.Precision.HIGHEST)


def _tc2(acc, g, dinv, b, a, w2):
    return pl.pallas_call(
        _tc2_body,
        grid=(GRID,),
        in_specs=[
            pl.BlockSpec((NC, BM, D), lambda i: (0, i, 0)),
            pl.BlockSpec((BM, D), lambda i: (i, 0)),
            pl.BlockSpec((BM, 1), lambda i: (i, 0)),
            pl.BlockSpec((1, D), lambda i: (0, 0)),
            pl.BlockSpec((1, D), lambda i: (0, 0)),
            pl.BlockSpec((D, D), lambda i: (0, 0)),
        ],
        out_specs=pl.BlockSpec((BM, D), lambda i: (i, 0)),
        out_shape=jax.ShapeDtypeStruct((N_PAD, D), jnp.float32),
    )(acc, g, dinv, b, a, w2)


def _tc3_body(acc_ref, g_ref, dinv_ref, b_ref, a_ref, out_ref):
    pre = (dinv_ref[...] * (acc_ref[0] + acc_ref[1] + g_ref[...])
           + b_ref[...])
    out_ref[...] = jnp.where(pre >= 0.0, pre, a_ref[...] * pre)


def _tc3(acc, g, dinv, b, a):
    return pl.pallas_call(
        _tc3_body,
        grid=(GRID,),
        in_specs=[
            pl.BlockSpec((NC, BM, D), lambda i: (0, i, 0)),
            pl.BlockSpec((BM, D), lambda i: (i, 0)),
            pl.BlockSpec((BM, 1), lambda i: (i, 0)),
            pl.BlockSpec((1, D), lambda i: (0, 0)),
            pl.BlockSpec((1, D), lambda i: (0, 0)),
        ],
        out_specs=pl.BlockSpec((BM, D), lambda i: (i, 0)),
        out_shape=jax.ShapeDtypeStruct((N_PAD, D), jnp.float32),
    )(acc, g, dinv, b, a)


def kernel(features, edge_index, edge_weight, W1, b1, a1, W2, b2, a2):
    src = edge_index[0]
    dst = edge_index[1]
    pad = E_PAD - E
    src_r = jnp.concatenate(
        [src, jnp.zeros((pad,), jnp.int32)]).reshape(NW, CHUNKS, CE)
    dst_r = jnp.concatenate(
        [dst, jnp.zeros((pad,), jnp.int32)]).reshape(NW, CHUNKS, CE)
    ew_r = jnp.concatenate(
        [edge_weight, jnp.zeros((pad,), jnp.float32)]).reshape(NW, CHUNKS, CE)
    pk = jnp.stack([src_r, dst_r], axis=2).reshape(TOTC, 2, CE)
    ew_f = ew_r.reshape(TOTC, CE)
    x = jnp.concatenate(
        [features, jnp.zeros((N_PAD - N, D), jnp.float32)], axis=0)

    deg2 = _deg_kernel(dst_r, ew_r).reshape(NC, N_PAD, 1)

    g1, dinv = _tc1(deg2, x, W1)
    acc1 = _agg_kernel(pk, ew_f, g1)
    g2 = _tc2(acc1, g1, dinv, b1.reshape(1, D), a1.reshape(1, D), W2)
    acc2 = _agg_kernel(pk, ew_f, g2)
    out = _tc3(acc2, g2, dinv, b2.reshape(1, D), a2.reshape(1, D))
    return out[:N]


# speedup vs baseline: 1.1387x; 1.1161x over previous
import functools

import jax
import jax.numpy as jnp
from jax import lax
from jax.experimental import pallas as pl
from jax.experimental.pallas import tpu as pltpu
from jax.experimental.pallas import tpu_sc as plsc

N = 10000
D = 128
E = 320000

NC = 2
NS = 16
NW = NC * NS

N_PAD = 10240
ROWS_PER_TILE = N_PAD // NS
E_PAD = 327680
CHUNKS = 80
CE = 128
TOTC = E_PAD // CE
CNT0 = 108
CNT1 = (TOTC - NS * CNT0) // NS

BM = 512
GRID = N_PAD // BM

_mesh = plsc.VectorSubcoreMesh(core_axis_name="c", subcore_axis_name="s")


@functools.partial(
    pl.kernel,
    out_type=jax.ShapeDtypeStruct((NC, N_PAD), jnp.float32),
    mesh=_mesh,
    scratch_types=[
        pltpu.VMEM((CHUNKS, CE), jnp.int32),
        pltpu.VMEM((CHUNKS, CE), jnp.float32),
        pltpu.VMEM((ROWS_PER_TILE,), jnp.float32),
        pltpu.VMEM_SHARED((N_PAD,), jnp.float32),
    ],
)
def _deg_kernel(dst_hbm, ew_hbm, out_hbm, dstb, ewb, zv, degsh):
    c = lax.axis_index("c")
    s = lax.axis_index("s")
    wid = c * NS + s
    z16 = jnp.zeros((16,), jnp.float32)
    for k in range(ROWS_PER_TILE // 16):
        zv[pl.ds(k * 16, 16)] = z16
    pltpu.sync_copy(zv, degsh.at[pl.ds(s * ROWS_PER_TILE, ROWS_PER_TILE)])
    plsc.subcore_barrier()
    pltpu.sync_copy(dst_hbm.at[wid], dstb)
    pltpu.sync_copy(ew_hbm.at[wid], ewb)

    def body(j, _):
        pltpu.sync_copy(ewb.at[j], degsh.at[dstb.at[j]], add=True)
        return 0

    lax.fori_loop(0, CHUNKS, body, 0)
    plsc.subcore_barrier()
    pltpu.sync_copy(degsh.at[pl.ds(s * ROWS_PER_TILE, ROWS_PER_TILE)],
                    out_hbm.at[c, pl.ds(s * ROWS_PER_TILE, ROWS_PER_TILE)])


@functools.partial(
    pl.kernel,
    out_type=jax.ShapeDtypeStruct((NC, N_PAD, D), jnp.float32),
    mesh=_mesh,
    scratch_types=[
        pltpu.VMEM((2, CE), jnp.int32),
        pltpu.VMEM((2, CE), jnp.int32),
        pltpu.VMEM((2, CE), jnp.int32),
        pltpu.VMEM((2, CE), jnp.int32),
        pltpu.VMEM((CE,), jnp.float32),
        pltpu.VMEM((CE,), jnp.float32),
        pltpu.VMEM((CE,), jnp.float32),
        pltpu.VMEM((CE,), jnp.float32),
        pltpu.VMEM((CE, D), jnp.float32),
        pltpu.VMEM((CE, D), jnp.float32),
        pltpu.VMEM_SHARED((N_PAD, D), jnp.float32),
        pltpu.SemaphoreType.DMA,
        pltpu.SemaphoreType.DMA,
        pltpu.SemaphoreType.DMA,
        pltpu.SemaphoreType.DMA,
        pltpu.SemaphoreType.DMA,
        pltpu.SemaphoreType.DMA,
        pltpu.SemaphoreType.DMA,
        pltpu.SemaphoreType.DMA,
        pltpu.SemaphoreType.DMA,
        pltpu.SemaphoreType.DMA,
    ],
)
def _agg_kernel(pk_hbm, ew_hbm, g_hbm, out_hbm,
                ib0, ib1, ib2, ib3, eb0, eb1, eb2, eb3, r0, r1, accsh,
                isem0, isem1, isem2, isem3, esem0, esem1, esem2, esem3,
                gsem0, gsem1):
    c = lax.axis_index("c")
    s = lax.axis_index("s")
    wid = c * NS + s
    z16 = jnp.zeros((16,), jnp.float32)
    ibs = (ib0, ib1, ib2, ib3)
    isems = (isem0, isem1, isem2, isem3)
    ebs = (eb0, eb1, eb2, eb3)
    esems = (esem0, esem1, esem2, esem3)
    rbs = (r0, r1)
    gsems = (gsem0, gsem1)

    def zrow(r, _):
        for cg in range(D // 16):
            r0[r, pl.ds(cg * 16, 16)] = z16
        return 0

    lax.fori_loop(0, CE, zrow, 0)
    for k in range(ROWS_PER_TILE // CE):
        pltpu.sync_copy(
            r0, accsh.at[pl.ds(s * ROWS_PER_TILE + k * CE, CE)])
    plsc.subcore_barrier()

    def scale(eb, rb):
        def grp(g2, _):
            for half in range(2):
                gi = g2 * 2 + half
                ew16 = eb[pl.ds(gi * 16, 16)]
                for lane in range(16):
                    e = gi * 16 + lane
                    w = ew16.at[jnp.full((16,), lane, jnp.int32)].get(
                        mode="promise_in_bounds")
                    for cg in range(D // 16):
                        sl = pl.ds(cg * 16, 16)
                        rb[e, sl] = rb[e, sl] * w
            return 0

        lax.fori_loop(0, CE // 32, grp, 0)

    def run(base, cnt):
        for b in range(4):
            pltpu.async_copy(pk_hbm.at[base + b], ibs[b], isems[b])
            pltpu.async_copy(ew_hbm.at[base + b], ebs[b], esems[b])
        pltpu.make_async_copy(pk_hbm.at[base], ib0, isem0).wait()
        pltpu.async_copy(g_hbm.at[ib0.at[0]], r0, gsem0)
        pltpu.make_async_copy(pk_hbm.at[base], ib1, isem1).wait()
        pltpu.async_copy(g_hbm.at[ib1.at[0]], r1, gsem1)

        def stage(j, i):
            ib = ibs[i]
            ib2_ = ibs[(i + 2) % 4]
            rb = rbs[i % 2]
            pltpu.make_async_copy(pk_hbm.at[base], ib2_,
                                  isems[(i + 2) % 4]).wait()
            pltpu.make_async_copy(ew_hbm.at[base], ebs[i], esems[i]).wait()
            pltpu.make_async_copy(g_hbm.at[ib.at[0]], rb,
                                  gsems[i % 2]).wait()
            scale(ebs[i], rb)
            pltpu.sync_copy(rb, accsh.at[ib.at[1]], add=True)
            pltpu.async_copy(g_hbm.at[ib2_.at[0]], rb, gsems[i % 2])
            nxt = base + jnp.minimum(j + 4, cnt - 1)
            pltpu.async_copy(pk_hbm.at[nxt], ib, isems[i])
            pltpu.async_copy(ew_hbm.at[nxt], ebs[i], esems[i])

        def body(j4, _):
            for i in range(4):
                stage(j4 * 4 + i, i)
            return 0

        lax.fori_loop(0, cnt // 4, body, 0)
        for i in ((cnt - 2) % 4, (cnt - 1) % 4):
            pltpu.make_async_copy(pk_hbm.at[base], ibs[i], isems[i]).wait()
        for i in (1, 2, 3, 0):
            pltpu.make_async_copy(ew_hbm.at[base], ebs[i], esems[i]).wait()
        pltpu.make_async_copy(g_hbm.at[ib0.at[0]], r0, gsem0).wait()
        pltpu.make_async_copy(g_hbm.at[ib1.at[0]], r1, gsem1).wait()

    @pl.when(c == 0)
    def _():
        run(s * CNT0, CNT0)

    @pl.when(c == 1)
    def _():
        run(NS * CNT0 + s * CNT1, CNT1)

    plsc.subcore_barrier()
    pltpu.sync_copy(accsh.at[pl.ds(s * ROWS_PER_TILE, ROWS_PER_TILE)],
                    out_hbm.at[c, pl.ds(s * ROWS_PER_TILE, ROWS_PER_TILE)])


def _tc1_body(deg_ref, x_ref, w_ref, g_ref, dinv_ref):
    deg = deg_ref[0] + deg_ref[1] + 1.0
    dinv = lax.rsqrt(deg)
    h = jnp.dot(x_ref[...], w_ref[...],
                preferred_element_type=jnp.float32,
                precision=lax.Precision.HIGHEST)
    g_ref[...] = dinv * h
    dinv_ref[...] = dinv


def _tc1(deg2, x, w1):
    return pl.pallas_call(
        _tc1_body,
        grid=(GRID,),
        in_specs=[
            pl.BlockSpec((NC, BM, 1), lambda i: (0, i, 0)),
            pl.BlockSpec((BM, D), lambda i: (i, 0)),
            pl.BlockSpec((D, D), lambda i: (0, 0)),
        ],
        out_specs=[
            pl.BlockSpec((BM, D), lambda i: (i, 0)),
            pl.BlockSpec((BM, 1), lambda i: (i, 0)),
        ],
        out_shape=[
            jax.ShapeDtypeStruct((N_PAD, D), jnp.float32),
            jax.ShapeDtypeStruct((N_PAD, 1), jnp.float32),
        ],
    )(deg2, x, w1)


def _tc2_body(acc_ref, g_ref, dinv_ref, b_ref, a_ref, w_ref, out_ref):
    dinv = dinv_ref[...]
    pre = dinv * (acc_ref[0] + acc_ref[1] + g_ref[...]) + b_ref[...]
    h = jnp.where(pre >= 0.0, pre, a_ref[...] * pre)
    out_ref[...] = dinv * jnp.dot(h, w_ref[...],
                                  preferred_element_type=jnp.float32,
                                  precision=lax.Precision.HIGHEST)


def _tc2(acc, g, dinv, b, a, w2):
    return pl.pallas_call(
        _tc2_body,
        grid=(GRID,),
        in_specs=[
            pl.BlockSpec((NC, BM, D), lambda i: (0, i, 0)),
            pl.BlockSpec((BM, D), lambda i: (i, 0)),
            pl.BlockSpec((BM, 1), lambda i: (i, 0)),
            pl.BlockSpec((1, D), lambda i: (0, 0)),
            pl.BlockSpec((1, D), lambda i: (0, 0)),
            pl.BlockSpec((D, D), lambda i: (0, 0)),
        ],
        out_specs=pl.BlockSpec((BM, D), lambda i: (i, 0)),
        out_shape=jax.ShapeDtypeStruct((N_PAD, D), jnp.float32),
    )(acc, g, dinv, b, a, w2)


def _tc3_body(acc_ref, g_ref, dinv_ref, b_ref, a_ref, out_ref):
    pre = (dinv_ref[...] * (acc_ref[0] + acc_ref[1] + g_ref[...])
           + b_ref[...])
    out_ref[...] = jnp.where(pre >= 0.0, pre, a_ref[...] * pre)


def _tc3(acc, g, dinv, b, a):
    return pl.pallas_call(
        _tc3_body,
        grid=(GRID,),
        in_specs=[
            pl.BlockSpec((NC, BM, D), lambda i: (0, i, 0)),
            pl.BlockSpec((BM, D), lambda i: (i, 0)),
            pl.BlockSpec((BM, 1), lambda i: (i, 0)),
            pl.BlockSpec((1, D), lambda i: (0, 0)),
            pl.BlockSpec((1, D), lambda i: (0, 0)),
        ],
        out_specs=pl.BlockSpec((BM, D), lambda i: (i, 0)),
        out_shape=jax.ShapeDtypeStruct((N_PAD, D), jnp.float32),
    )(acc, g, dinv, b, a)


def kernel(features, edge_index, edge_weight, W1, b1, a1, W2, b2, a2):
    src = edge_index[0]
    dst = edge_index[1]
    pad = E_PAD - E
    src_r = jnp.concatenate(
        [src, jnp.zeros((pad,), jnp.int32)]).reshape(NW, CHUNKS, CE)
    dst_r = jnp.concatenate(
        [dst, jnp.zeros((pad,), jnp.int32)]).reshape(NW, CHUNKS, CE)
    ew_r = jnp.concatenate(
        [edge_weight, jnp.zeros((pad,), jnp.float32)]).reshape(NW, CHUNKS, CE)
    pk = jnp.stack([src_r, dst_r], axis=2).reshape(TOTC, 2, CE)
    ew_f = ew_r.reshape(TOTC, CE)
    x = jnp.concatenate(
        [features, jnp.zeros((N_PAD - N, D), jnp.float32)], axis=0)

    deg2 = _deg_kernel(dst_r, ew_r).reshape(NC, N_PAD, 1)

    g1, dinv = _tc1(deg2, x, W1)
    acc1 = _agg_kernel(pk, ew_f, g1)
    g2 = _tc2(acc1, g1, dinv, b1.reshape(1, D), a1.reshape(1, D), W2)
    acc2 = _agg_kernel(pk, ew_f, g2)
    out = _tc3(acc2, g2, dinv, b2.reshape(1, D), a2.reshape(1, D))
    return out[:N]
